# Initial kernel scaffold; baseline (speedup 1.0000x reference)
import jax
import jax.numpy as jnp
from jax.experimental import pallas as pl


def kernel(s, x_cluster, a, b):
    raise NotImplementedError("write your pallas kernel here")


# speedup vs baseline: 1.2258x; 1.2258x over previous
import functools

import jax
import jax.numpy as jnp
from jax import lax
from jax.experimental import pallas as pl
from jax.experimental.pallas import tpu as pltpu
from jax.experimental.pallas import tpu_sc as plsc

BATCH = 16384
NC = 2
NS = 16
NW = NC * NS
LANES = 16
B_PER_W = BATCH // NW
CHUNK = 128
NCHUNK = B_PER_W // CHUNK

_mesh = plsc.VectorSubcoreMesh(core_axis_name="c", subcore_axis_name="s")


@functools.partial(
    pl.kernel,
    out_type=jax.ShapeDtypeStruct((NW, B_PER_W), jnp.float32),
    mesh=_mesh,
    scratch_types=[
        pltpu.VMEM((NCHUNK, CHUNK), jnp.int32),
        pltpu.VMEM((B_PER_W,), jnp.float32),
        pltpu.VMEM((B_PER_W,), jnp.float32),
        pltpu.VMEM((B_PER_W,), jnp.float32),
        pltpu.VMEM((B_PER_W,), jnp.float32),
        pltpu.SemaphoreType.DMA,
    ],
)
def _sn_sc_kernel(s_hbm, idx_hbm, a_hbm, b_hbm, out_hbm,
                  idx_v, s_v, ga_v, gb_v, o_v, sem):
    wid = lax.axis_index("s") * NC + lax.axis_index("c")

    pltpu.sync_copy(idx_hbm.at[wid], idx_v)
    pltpu.sync_copy(s_hbm.at[wid], s_v)

    copies = []
    for j in range(NCHUNK):
        copies.append(pltpu.async_copy(
            a_hbm.at[idx_v.at[j]], ga_v.at[pl.ds(j * CHUNK, CHUNK)], sem))
        copies.append(pltpu.async_copy(
            b_hbm.at[idx_v.at[j]], gb_v.at[pl.ds(j * CHUNK, CHUNK)], sem))
    for c in copies:
        c.wait()

    for i in range(B_PER_W // LANES):
        sl = pl.ds(i * LANES, LANES)
        o_v[sl] = jnp.abs(gb_v[sl]) - jnp.abs(ga_v[sl]) * s_v[sl]

    pltpu.sync_copy(o_v, out_hbm.at[wid])


def kernel(s, x_cluster, a, b):
    idx = x_cluster.astype(jnp.int32).reshape(NW, NCHUNK, CHUNK)
    s2 = s.reshape(NW, B_PER_W)
    out = _sn_sc_kernel(s2, idx, a, b)
    return out.reshape(BATCH)


# speedup vs baseline: 1.2683x; 1.0347x over previous
import functools

import jax
import jax.numpy as jnp
from jax import lax
from jax.experimental import pallas as pl
from jax.experimental.pallas import tpu as pltpu
from jax.experimental.pallas import tpu_sc as plsc

BATCH = 16384
NC = 2
NS = 16
NW = NC * NS
LANES = 16
B_PER_W = BATCH // NW
CHUNK = 128
NCHUNK = B_PER_W // CHUNK

_mesh = plsc.VectorSubcoreMesh(core_axis_name="c", subcore_axis_name="s")


@functools.partial(
    pl.kernel,
    out_type=jax.ShapeDtypeStruct((NW, B_PER_W), jnp.float32),
    mesh=_mesh,
    scratch_types=[
        pltpu.VMEM((NCHUNK, CHUNK), jnp.int32),
        pltpu.VMEM((B_PER_W,), jnp.float32),
        pltpu.VMEM((B_PER_W,), jnp.float32),
        pltpu.VMEM((B_PER_W,), jnp.float32),
        pltpu.VMEM((B_PER_W,), jnp.float32),
        pltpu.SemaphoreType.DMA,
        pltpu.SemaphoreType.DMA,
        pltpu.SemaphoreType.DMA,
        pltpu.SemaphoreType.DMA,
        pltpu.SemaphoreType.DMA,
        pltpu.SemaphoreType.DMA,
    ],
)
def _sn_sc_kernel(s_hbm, idx_hbm, a_hbm, b_hbm, out_hbm,
                  idx_v, s_v, ga_v, gb_v, o_v,
                  g0, g1, g2, g3, ssem, osem):
    wid = lax.axis_index("s") * NC + lax.axis_index("c")
    gsem = (g0, g1, g2, g3)

    pltpu.sync_copy(idx_hbm.at[wid], idx_v)
    s_cp = pltpu.async_copy(s_hbm.at[wid], s_v, ssem)

    copies = []
    for j in range(NCHUNK):
        copies.append((
            pltpu.async_copy(
                a_hbm.at[idx_v.at[j]], ga_v.at[pl.ds(j * CHUNK, CHUNK)],
                gsem[j]),
            pltpu.async_copy(
                b_hbm.at[idx_v.at[j]], gb_v.at[pl.ds(j * CHUNK, CHUNK)],
                gsem[j]),
        ))
    s_cp.wait()

    out_copies = []
    for j in range(NCHUNK):
        ca, cb = copies[j]
        ca.wait()
        cb.wait()
        for i in range(CHUNK // LANES):
            sl = pl.ds(j * CHUNK + i * LANES, LANES)
            o_v[sl] = jnp.abs(gb_v[sl]) - jnp.abs(ga_v[sl]) * s_v[sl]
        out_copies.append(pltpu.async_copy(
            o_v.at[pl.ds(j * CHUNK, CHUNK)],
            out_hbm.at[wid, pl.ds(j * CHUNK, CHUNK)], osem))
    for c in out_copies:
        c.wait()


def kernel(s, x_cluster, a, b):
    idx = x_cluster.astype(jnp.int32).reshape(NW, NCHUNK, CHUNK)
    s2 = s.reshape(NW, B_PER_W)
    out = _sn_sc_kernel(s2, idx, a, b)
    return out.reshape(BATCH)


# speedup vs baseline: 1.3425x; 1.0585x over previous
import functools

import jax
import jax.numpy as jnp
from jax import lax
from jax.experimental import pallas as pl
from jax.experimental.pallas import tpu as pltpu
from jax.experimental.pallas import tpu_sc as plsc

BATCH = 16384
NC = 2
NS = 16
NW = NC * NS
LANES = 16
B_PER_W = BATCH // NW
CHUNK = 128
NCHUNK = B_PER_W // CHUNK

_mesh = plsc.VectorSubcoreMesh(core_axis_name="c", subcore_axis_name="s")


@functools.partial(
    pl.kernel,
    out_type=jax.ShapeDtypeStruct((BATCH,), jnp.float32),
    mesh=_mesh,
    scratch_types=[
        pltpu.VMEM((B_PER_W,), jnp.int32),
        pltpu.VMEM((B_PER_W,), jnp.float32),
        pltpu.VMEM((B_PER_W,), jnp.float32),
        pltpu.VMEM((B_PER_W,), jnp.float32),
        pltpu.VMEM((B_PER_W,), jnp.float32),
        pltpu.SemaphoreType.DMA,
        pltpu.SemaphoreType.DMA,
        pltpu.SemaphoreType.DMA,
        pltpu.SemaphoreType.DMA,
        pltpu.SemaphoreType.DMA,
        pltpu.SemaphoreType.DMA,
    ],
)
def _sn_sc_kernel(s_hbm, idx_hbm, a_hbm, b_hbm, out_hbm,
                  idx_v, s_v, ga_v, gb_v, o_v,
                  g0, g1, g2, g3, ssem, osem):
    wid = lax.axis_index("s") * NC + lax.axis_index("c")
    base = wid * B_PER_W
    gsem = (g0, g1, g2, g3)

    pltpu.sync_copy(idx_hbm.at[pl.ds(base, B_PER_W)], idx_v)
    s_cp = pltpu.async_copy(s_hbm.at[pl.ds(base, B_PER_W)], s_v, ssem)

    copies = []
    for j in range(NCHUNK):
        isl = pl.ds(j * CHUNK, CHUNK)
        copies.append((
            pltpu.async_copy(a_hbm.at[idx_v.at[isl]], ga_v.at[isl], gsem[j]),
            pltpu.async_copy(b_hbm.at[idx_v.at[isl]], gb_v.at[isl], gsem[j]),
        ))
    s_cp.wait()

    out_copies = []
    for j in range(NCHUNK):
        ca, cb = copies[j]
        ca.wait()
        cb.wait()
        for i in range(CHUNK // LANES):
            sl = pl.ds(j * CHUNK + i * LANES, LANES)
            o_v[sl] = jnp.abs(gb_v[sl]) - jnp.abs(ga_v[sl]) * s_v[sl]
        out_copies.append(pltpu.async_copy(
            o_v.at[pl.ds(j * CHUNK, CHUNK)],
            out_hbm.at[pl.ds(base + j * CHUNK, CHUNK)], osem))
    for c in out_copies:
        c.wait()


def kernel(s, x_cluster, a, b):
    return _sn_sc_kernel(s, x_cluster.astype(jnp.int32), a, b)
